# speedup vs baseline: 1.2930x; 1.2930x over previous
import functools

import jax
import jax.numpy as jnp
from jax.experimental import pallas as pl
from jax.experimental.pallas import tpu as pltpu

N_GRAPHS = 100


def _proj_kernel(x_ref, wl_ref, bl_ref, wr_ref, br_ref, hl_ref, hr_ref):
    x = x_ref[...]
    hl_ref[...] = x @ wl_ref[...] + bl_ref[...]
    hr_ref[...] = x @ wr_ref[...] + br_ref[...]


def _proj(x, p):
    n, cin = x.shape
    cout = p["Wl"].shape[1]
    out_sd = jax.ShapeDtypeStruct((n, cout), jnp.float32)
    return pl.pallas_call(
        _proj_kernel,
        out_shape=(out_sd, out_sd),
    )(x, p["Wl"], p["bl"][None, :], p["Wr"], p["br"][None, :])


def _gatv2(x, src, dst, p, n):
    hl, hr = _proj(x, p)
    xj = hl[src]
    xi = hr[dst]
    e = jax.nn.leaky_relu(xj + xi, 0.2)
    logit = jnp.sum(e * p["att"], axis=-1)
    s = jnp.sum(jax.nn.leaky_relu(hl + hr, 0.2) * p["att"], axis=-1)
    w = jnp.exp(logit - s[dst])
    num = jax.ops.segment_sum(xj * w[:, None], dst, num_segments=n)
    den = jax.ops.segment_sum(w, dst, num_segments=n)
    return num / (den[:, None] + 1e-16) + p["b"]


def _scatter_mean(srcv, index, num_segments):
    s = jax.ops.segment_sum(srcv, index, num_segments=num_segments)
    cnt = jax.ops.segment_sum(jnp.ones((srcv.shape[0],), srcv.dtype), index,
                              num_segments=num_segments)
    return s / jnp.clip(cnt, 1, None)[:, None]


def kernel(el, lat, batch, edge_index, params):
    n = lat.shape[0]
    loops = jnp.arange(n, dtype=edge_index.dtype)
    src = jnp.concatenate([edge_index[0], loops])
    dst = jnp.concatenate([edge_index[1], loops])
    silu = jax.nn.silu
    x_el = params["emb"][el]
    x_el = silu(_gatv2(x_el, src, dst, params["element"], n))
    x_lat = silu(_gatv2(lat, src, dst, params["latent"], n))
    x = jnp.concatenate([x_el, x_lat], axis=-1)
    x = silu(_gatv2(x, src, dst, params["pre"], n))
    nblk = len(params["coords"])
    for i, p in enumerate(params["coords"]):
        h = _gatv2(x, src, dst, p, n)
        x = jax.nn.sigmoid(h) if i == nblk - 1 else silu(h)
    coords = x
    lengths = silu(_gatv2(coords, src, dst, params["post_len"], n))
    for p in params["len_blocks"]:
        lengths = silu(_gatv2(lengths, src, dst, p, n))
    lengths = _scatter_mean(lengths, batch, N_GRAPHS)
    angles = silu(_gatv2(coords, src, dst, params["post_ang"], n))
    for p in params["ang_blocks"]:
        angles = silu(_gatv2(angles, src, dst, p, n))
    angles = _scatter_mean(angles, batch, N_GRAPHS)
    return coords, lengths, angles


# speedup vs baseline: 2.8297x; 2.1885x over previous
import functools

import jax
import jax.numpy as jnp
from jax import lax
from jax.experimental import pallas as pl
from jax.experimental.pallas import tpu as pltpu
from jax.experimental.pallas import tpu_sc as plsc

N_GRAPHS = 100
N = 10000
NP = 10016
DUMMY = 10008
E = 330000
NW = 32
EPW = 10320
E_PAD = NW * EPW
CHUNK = 2064
NEG_SLOPE = 0.2


def _proj_kernel(x_ref, wl_ref, bl_ref, wr_ref, br_ref, hl_ref, hr_ref):
    x = x_ref[...]
    hl_ref[...] = x @ wl_ref[...] + bl_ref[...]
    hr_ref[...] = x @ wr_ref[...] + br_ref[...]


def _proj(x, p):
    n = x.shape[0]
    cout = p["Wl"].shape[1]
    out_sd = jax.ShapeDtypeStruct((n, cout), jnp.float32)
    return pl.pallas_call(
        _proj_kernel,
        out_shape=(out_sd, out_sd),
    )(x, p["Wl"], p["bl"][None, :], p["Wr"], p["br"][None, :])


def _lrelu(z):
    return jnp.where(z >= 0, z, z * NEG_SLOPE)


def _silu(z):
    return z * jax.nn.sigmoid(z)


def _tbl_from_x_kernel(x_ref, wlt_ref, wrt_ref, bl_ref, br_ref, att_ref, tbl_ref):
    x = x_ref[...]
    hl = wlt_ref[...] @ x + bl_ref[...]
    hr = wrt_ref[...] @ x + br_ref[...]
    att = att_ref[...]
    s = jnp.sum(_lrelu(hl + hr) * att, axis=0, keepdims=True)
    e3 = (jax.lax.broadcasted_iota(jnp.int32, (4, 1), 0) == 3).astype(jnp.float32)
    hrs = hr * (1.0 - e3) + s * e3
    tbl_ref[...] = jnp.concatenate([hl, hrs], axis=0)


def _tbl_from_x(xt, p):
    wlt = jnp.zeros((4, 4), jnp.float32).at[:3, :3].set(p["Wl"].T)
    wrt = jnp.zeros((4, 4), jnp.float32).at[:3, :3].set(p["Wr"].T)
    bl = jnp.zeros((4, 1), jnp.float32).at[:3, 0].set(p["bl"])
    br = jnp.zeros((4, 1), jnp.float32).at[:3, 0].set(p["br"])
    att = jnp.zeros((4, 1), jnp.float32).at[:3, 0].set(p["att"])
    return pl.pallas_call(
        _tbl_from_x_kernel,
        out_shape=jax.ShapeDtypeStruct((8, NP), jnp.float32),
    )(xt, wlt, wrt, bl, br, att)


def _finalize_kernel(parts_ref, b_ref, x_ref):
    summed = jnp.sum(parts_ref[...], axis=0)
    den = summed[3:4, :]
    h = summed / (den + 1e-16) + b_ref[...]
    x_ref[...] = _silu(h)


def _finalize(parts, b):
    bp = jnp.zeros((4, 1), jnp.float32).at[:3, 0].set(b)
    return pl.pallas_call(
        _finalize_kernel,
        out_shape=jax.ShapeDtypeStruct((4, NP), jnp.float32),
    )(parts, bp)


def _sc_edge3_body(tbl_hbm, src_hbm, dst_hbm, att_hbm, out_hbm,
                   tbl_v, acc_v, src_v, dst_v, att_v):
    wid = lax.axis_index("s") * 2 + lax.axis_index("c")

    def zbody(i, c):
        acc_v[pl.ds(i * 16, 16)] = jnp.zeros((16,), jnp.float32)
        return c
    lax.fori_loop(0, (NP * 4) // 16, zbody, 0)

    pltpu.sync_copy(tbl_hbm, tbl_v)
    pltpu.sync_copy(att_hbm, att_v)
    a0 = att_v[pl.ds(0, 16)]
    a1 = att_v[pl.ds(16, 16)]
    a2 = att_v[pl.ds(32, 16)]

    base = wid * EPW

    def group(g, c):
        s16 = src_v[pl.ds(g * 16, 16)]
        d16 = dst_v[pl.ds(g * 16, 16)]
        x0 = plsc.load_gather(tbl_v, [s16])
        x1 = plsc.load_gather(tbl_v, [s16 + NP])
        x2 = plsc.load_gather(tbl_v, [s16 + 2 * NP])
        y0 = plsc.load_gather(tbl_v, [d16 + 4 * NP])
        y1 = plsc.load_gather(tbl_v, [d16 + 5 * NP])
        y2 = plsc.load_gather(tbl_v, [d16 + 6 * NP])
        sv = plsc.load_gather(tbl_v, [d16 + 7 * NP])
        logit = (a0 * _lrelu(x0 + y0) + a1 * _lrelu(x1 + y1)
                 + a2 * _lrelu(x2 + y2))
        w = jnp.exp(logit - sv)
        plsc.addupdate_scatter(acc_v, [d16], w * x0)
        plsc.addupdate_scatter(acc_v, [d16 + NP], w * x1)
        plsc.addupdate_scatter(acc_v, [d16 + 2 * NP], w * x2)
        plsc.addupdate_scatter(acc_v, [d16 + 3 * NP], w)
        return c

    for k in range(EPW // CHUNK):
        pltpu.sync_copy(src_hbm.at[pl.ds(base + k * CHUNK, CHUNK)], src_v)
        pltpu.sync_copy(dst_hbm.at[pl.ds(base + k * CHUNK, CHUNK)], dst_v)
        lax.fori_loop(0, CHUNK // 16, group, 0)

    pltpu.sync_copy(acc_v, out_hbm.at[wid])


_sc_edge3 = functools.partial(
    pl.kernel,
    out_type=jax.ShapeDtypeStruct((NW, NP * 4), jnp.float32),
    mesh=plsc.VectorSubcoreMesh(core_axis_name="c", subcore_axis_name="s"),
    compiler_params=pltpu.CompilerParams(needs_layout_passes=False),
    scratch_types=[
        pltpu.VMEM((NP * 8,), jnp.float32),
        pltpu.VMEM((NP * 4,), jnp.float32),
        pltpu.VMEM((CHUNK,), jnp.int32),
        pltpu.VMEM((CHUNK,), jnp.int32),
        pltpu.VMEM((48,), jnp.float32),
    ],
)(_sc_edge3_body)


def _gat3_parts(tbl, srcp, dstp, p):
    att48 = jnp.repeat(p["att"].astype(jnp.float32), 16)
    parts = _sc_edge3(tbl.reshape(-1), srcp, dstp, att48)
    return parts.reshape(NW, 4, NP)


def _gatv2_xla(x, src, dst, p, n):
    hl, hr = _proj(x, p)
    xj = hl[src]
    logit = jnp.sum(jax.nn.leaky_relu(xj + hr[dst], NEG_SLOPE) * p["att"], axis=-1)
    s = jnp.sum(jax.nn.leaky_relu(hl + hr, NEG_SLOPE) * p["att"], axis=-1)
    w = jnp.exp(logit - s[dst])
    num = jax.ops.segment_sum(xj * w[:, None], dst, num_segments=n)
    den = jax.ops.segment_sum(w, dst, num_segments=n)
    return num / (den[:, None] + 1e-16) + p["b"]


def _scatter_mean(srcv, index, num_segments):
    s = jax.ops.segment_sum(srcv, index, num_segments=num_segments)
    cnt = jax.ops.segment_sum(jnp.ones((srcv.shape[0],), srcv.dtype), index,
                              num_segments=num_segments)
    return s / jnp.clip(cnt, 1, None)[:, None]


def _chain3(xt, srcp, dstp, plist):
    tbl = _tbl_from_x(xt, plist[0])
    for i, p in enumerate(plist):
        parts = _gat3_parts(tbl, srcp, dstp, p)
        if i + 1 < len(plist):
            xnext = _finalize(parts, p["b"])
            tbl = _tbl_from_x(xnext, plist[i + 1])
        else:
            return _finalize(parts, p["b"])


def kernel(el, lat, batch, edge_index, params):
    n = N
    loops = jnp.arange(n, dtype=edge_index.dtype)
    src = jnp.concatenate([edge_index[0], loops])
    dst = jnp.concatenate([edge_index[1], loops])
    pad = jnp.full((E_PAD - E,), DUMMY, src.dtype)
    srcp = jnp.concatenate([src, pad])
    dstp = jnp.concatenate([dst, pad])

    x_el = params["emb"][el]
    x_el = _silu(_gatv2_xla(x_el, src, dst, params["element"], n))
    x_lat = _silu(_gatv2_xla(lat, src, dst, params["latent"], n))
    x = jnp.concatenate([x_el, x_lat], axis=-1)
    x = _silu(_gatv2_xla(x, src, dst, params["pre"], n))
    nblk = len(params["coords"])
    for i, p in enumerate(params["coords"]):
        h = _gatv2_xla(x, src, dst, p, n)
        x = jax.nn.sigmoid(h) if i == nblk - 1 else _silu(h)
    coords = x

    xt = jnp.zeros((4, NP), jnp.float32).at[:3, :N].set(coords.T)
    lengths = _chain3(xt, srcp, dstp,
                      [params["post_len"]] + list(params["len_blocks"]))
    lengths = _scatter_mean(lengths[:3, :N].T, batch, N_GRAPHS)
    angles = _chain3(xt, srcp, dstp,
                     [params["post_ang"]] + list(params["ang_blocks"]))
    angles = _scatter_mean(angles[:3, :N].T, batch, N_GRAPHS)
    return coords, lengths, angles


# speedup vs baseline: 3.4914x; 1.2338x over previous
import functools

import jax
import jax.numpy as jnp
from jax import lax
from jax.experimental import pallas as pl
from jax.experimental.pallas import tpu as pltpu
from jax.experimental.pallas import tpu_sc as plsc

N_GRAPHS = 100
N = 10000
NP = 10016
DUMMY = 10008
E = 330000
NW = 32
NS = 16
EPW = 10368
E_PAD = NW * EPW
CHUNK3 = 2592
CP = 128
K1 = 128
NKC1 = EPW // K1
K2 = 96
NKC2 = EPW // K2
WHALF = EPW // 2
RPT = 632
RPT_LAST = NP - (NS - 1) * RPT
NEG_SLOPE = 0.2


def _lrelu(z):
    return jnp.maximum(z, z * NEG_SLOPE)


def _silu(z):
    return z * jax.nn.sigmoid(z)


_MESH = plsc.VectorSubcoreMesh(core_axis_name="c", subcore_axis_name="s")
_SC_PARAMS = pltpu.CompilerParams(needs_layout_passes=False)
_TC_PARAMS = pltpu.CompilerParams(vmem_limit_bytes=60 * 1024 * 1024)


def _tbl_from_x_kernel(x_ref, wlt_ref, wrt_ref, bl_ref, br_ref, att_ref, tbl_ref):
    x = x_ref[...]
    hl = wlt_ref[...] @ x + bl_ref[...]
    hr = wrt_ref[...] @ x + br_ref[...]
    att = att_ref[...]
    s = jnp.sum(_lrelu(hl + hr) * att, axis=0, keepdims=True)
    e3 = (jax.lax.broadcasted_iota(jnp.int32, (4, 1), 0) == 3).astype(jnp.float32)
    hrs = hr * (1.0 - e3) + s * e3
    tbl_ref[...] = jnp.concatenate([hl, hrs], axis=0)


def _tbl_from_x(xt, p):
    cin = p["Wl"].shape[0]
    cin_p = xt.shape[0]
    wlt = jnp.zeros((4, cin_p), jnp.float32).at[:3, :cin].set(p["Wl"].T)
    wrt = jnp.zeros((4, cin_p), jnp.float32).at[:3, :cin].set(p["Wr"].T)
    bl = jnp.zeros((4, 1), jnp.float32).at[:3, 0].set(p["bl"])
    br = jnp.zeros((4, 1), jnp.float32).at[:3, 0].set(p["br"])
    att = jnp.zeros((4, 1), jnp.float32).at[:3, 0].set(p["att"])
    return pl.pallas_call(
        _tbl_from_x_kernel,
        out_shape=jax.ShapeDtypeStruct((8, NP), jnp.float32),
    )(xt, wlt, wrt, bl, br, att)


def _finalize_kernel(sig, parts_ref, b_ref, x_ref):
    summed = jnp.sum(parts_ref[...], axis=0)
    den = summed[3:4, :]
    h = summed / (den + 1e-16) + b_ref[...]
    x_ref[...] = jax.nn.sigmoid(h) if sig else _silu(h)


def _finalize(parts, b, sig=False):
    bp = jnp.zeros((4, 1), jnp.float32).at[:3, 0].set(b)
    return pl.pallas_call(
        functools.partial(_finalize_kernel, sig),
        out_shape=jax.ShapeDtypeStruct((4, NP), jnp.float32),
    )(parts, bp)


def _sc_edge3_body(tbl_hbm, src_hbm, dst_hbm, att_hbm, out_hbm,
                   tbl_v, acc_v, src_v, dst_v, att_v):
    wid = lax.axis_index("s") * 2 + lax.axis_index("c")

    def zbody(i, c):
        acc_v[pl.ds(i * 16, 16)] = jnp.zeros((16,), jnp.float32)
        return c
    lax.fori_loop(0, (NP * 4) // 16, zbody, 0)

    pltpu.sync_copy(tbl_hbm, tbl_v)
    pltpu.sync_copy(att_hbm, att_v)
    a0 = att_v[pl.ds(0, 16)]
    a1 = att_v[pl.ds(16, 16)]
    a2 = att_v[pl.ds(32, 16)]

    base = wid * EPW

    def group(g, c):
        s16 = src_v[pl.ds(g * 16, 16)]
        d16 = dst_v[pl.ds(g * 16, 16)]
        x0 = plsc.load_gather(tbl_v, [s16])
        x1 = plsc.load_gather(tbl_v, [s16 + NP])
        x2 = plsc.load_gather(tbl_v, [s16 + 2 * NP])
        y0 = plsc.load_gather(tbl_v, [d16 + 4 * NP])
        y1 = plsc.load_gather(tbl_v, [d16 + 5 * NP])
        y2 = plsc.load_gather(tbl_v, [d16 + 6 * NP])
        sv = plsc.load_gather(tbl_v, [d16 + 7 * NP])
        logit = (a0 * _lrelu(x0 + y0) + a1 * _lrelu(x1 + y1)
                 + a2 * _lrelu(x2 + y2))
        w = jnp.exp(logit - sv)
        plsc.addupdate_scatter(acc_v, [d16], w * x0)
        plsc.addupdate_scatter(acc_v, [d16 + NP], w * x1)
        plsc.addupdate_scatter(acc_v, [d16 + 2 * NP], w * x2)
        plsc.addupdate_scatter(acc_v, [d16 + 3 * NP], w)
        return c

    for k in range(EPW // CHUNK3):
        pltpu.sync_copy(src_hbm.at[pl.ds(base + k * CHUNK3, CHUNK3)], src_v)
        pltpu.sync_copy(dst_hbm.at[pl.ds(base + k * CHUNK3, CHUNK3)], dst_v)
        lax.fori_loop(0, CHUNK3 // 16, group, 0)

    pltpu.sync_copy(acc_v, out_hbm.at[wid])


_sc_edge3 = functools.partial(
    pl.kernel,
    out_type=jax.ShapeDtypeStruct((NW, NP * 4), jnp.float32),
    mesh=_MESH,
    compiler_params=_SC_PARAMS,
    scratch_types=[
        pltpu.VMEM((NP * 8,), jnp.float32),
        pltpu.VMEM((NP * 4,), jnp.float32),
        pltpu.VMEM((CHUNK3,), jnp.int32),
        pltpu.VMEM((CHUNK3,), jnp.int32),
        pltpu.VMEM((48,), jnp.float32),
    ],
)(_sc_edge3_body)


def _gat3_parts(tbl, srcp, dstp, p):
    att48 = jnp.repeat(p["att"].astype(jnp.float32), 16)
    parts = _sc_edge3(tbl.reshape(-1), srcp, dstp, att48)
    return parts.reshape(NW, 4, NP)


def _chain3(xt, srcp, dstp, plist):
    tbl = _tbl_from_x(xt, plist[0])
    for i, p in enumerate(plist):
        parts = _gat3_parts(tbl, srcp, dstp, p)
        if i + 1 < len(plist):
            xnext = _finalize(parts, p["b"])
            tbl = _tbl_from_x(xnext, plist[i + 1])
        else:
            return _finalize(parts, p["b"])


def _proj_wide_body(nch, x_ref, wl_ref, bl_ref, wr_ref, br_ref, att_ref,
                    *out_refs):
    x = x_ref[...]
    hl = x @ wl_ref[...] + bl_ref[...]
    hr = x @ wr_ref[...] + br_ref[...]
    s = jnp.sum(_lrelu(hl + hr) * att_ref[...], axis=-1, keepdims=True)
    for i in range(nch):
        out_refs[i][...] = hl[:, i * CP:(i + 1) * CP]
        out_refs[nch + i][...] = hr[:, i * CP:(i + 1) * CP]
    out_refs[2 * nch][...] = s[:, 0]


def _proj_wide(x, p, nch):
    cin = p["Wl"].shape[0]
    c = p["Wl"].shape[1]
    cpad = nch * CP
    wl = jnp.zeros((cin, cpad), jnp.float32).at[:, :c].set(p["Wl"])
    wr = jnp.zeros((cin, cpad), jnp.float32).at[:, :c].set(p["Wr"])
    bl = jnp.zeros((1, cpad), jnp.float32).at[0, :c].set(p["bl"])
    br = jnp.zeros((1, cpad), jnp.float32).at[0, :c].set(p["br"])
    att = jnp.zeros((1, cpad), jnp.float32).at[0, :c].set(p["att"])
    outs = ([jax.ShapeDtypeStruct((NP, CP), jnp.float32)] * (2 * nch)
            + [jax.ShapeDtypeStruct((NP,), jnp.float32)])
    return pl.pallas_call(
        functools.partial(_proj_wide_body, nch),
        out_shape=tuple(outs),
        compiler_params=_TC_PARAMS,
    )(x, wl, bl, wr, br, att), att[0]


def _fin_wide_body(nch, c, act, *refs):
    num_refs = refs[:nch]
    den_ref, b_ref = refs[nch], refs[nch + 1]
    x_ref = refs[nch + 2]
    num = jnp.concatenate([r[...][0] + r[...][1] for r in num_refs], axis=-1)
    den = jnp.sum(den_ref[...], axis=0)[:, None]
    h = num[:, :c] / (den + 1e-16) + b_ref[...]
    x_ref[...] = _silu(h) if act == "silu" else h


def _fin_wide(numparts, denparts, b, c, act="silu"):
    return pl.pallas_call(
        functools.partial(_fin_wide_body, len(numparts), c, act),
        out_shape=jax.ShapeDtypeStruct((NP, c), jnp.float32),
        compiler_params=_TC_PARAMS,
    )(*numparts, denparts, b[None, :])


def _sc_wide_p1_body(nch, *refs):
    hl_hbm = refs[:nch]
    hr_hbm = refs[nch:2 * nch]
    s_hbm, src_hbm, dst_hbm, att_hbm = refs[2 * nch:2 * nch + 4]
    w_hbm, den_hbm = refs[2 * nch + 4:2 * nch + 6]
    (rows_hl, rows_hr, s_all, w_v, den_v, src2, dst2,
     att_v) = refs[2 * nch + 6:]

    cid = lax.axis_index("c")
    sid = lax.axis_index("s")
    wid = sid * 2 + cid

    def zden(i, c):
        den_v[pl.ds(i * 16, 16)] = jnp.zeros((16,), jnp.float32)
        return c
    lax.fori_loop(0, NP // 16, zden, 0)
    pltpu.sync_copy(att_hbm, att_v)
    pltpu.sync_copy(s_hbm, s_all)
    pltpu.sync_copy(src_hbm.at[wid], src2)
    pltpu.sync_copy(dst_hbm.at[wid], dst2)

    iota = lax.iota(jnp.int32, 16)
    row16 = [iota + g * 16 for g in range(K1 // 16)]

    def pass1_chunk(k, c):
        carry = tuple(jnp.zeros((16,), jnp.float32) for _ in range(K1 // 16))
        for i in range(nch):
            pltpu.sync_copy(hl_hbm[i].at[src2.at[k]], rows_hl)
            pltpu.sync_copy(hr_hbm[i].at[dst2.at[k]], rows_hr)

            def cbody(ch, carry, _i=i):
                colc = jnp.full((16,), ch, jnp.int32)
                aspl = plsc.load_gather(att_v, [colc + _i * CP])
                out = []
                for g in range(K1 // 16):
                    a = plsc.load_gather(rows_hl, [row16[g], colc])
                    b = plsc.load_gather(rows_hr, [row16[g], colc])
                    l = _lrelu(a + b)
                    out.append(carry[g] + aspl * l)
                return tuple(out)
            carry = lax.fori_loop(0, CP, cbody, carry)

        for g in range(K1 // 16):
            d16 = dst2[k, pl.ds(g * 16, 16)]
            sg = plsc.load_gather(s_all, [d16])
            w16 = jnp.exp(carry[g] - sg)
            plsc.addupdate_scatter(den_v, [d16], w16)
            idx = k * K1 + g * 16
            h = idx // WHALF
            w_v[h, pl.ds(idx - h * WHALF, 16)] = w16
        return c

    lax.fori_loop(0, NKC1, pass1_chunk, 0)
    pltpu.sync_copy(w_v.at[0], w_hbm.at[0, wid])
    pltpu.sync_copy(w_v.at[1], w_hbm.at[1, wid])
    pltpu.sync_copy(den_v, den_hbm.at[wid])


def _sc_wide_p2_body(nch, *refs):
    hl_hbm = refs[:nch]
    src_hbm, dst_hbm, w_hbm, zer_hbm = refs[nch:nch + 4]
    num_hbm = refs[nch + 4:2 * nch + 4]
    rows_hl, w_v, src2, dst2, acc_sh = refs[2 * nch + 4:]

    cid = lax.axis_index("c")
    sid = lax.axis_index("s")
    wid = sid * 2 + cid

    pltpu.sync_copy(src_hbm.at[wid], src2)
    pltpu.sync_copy(dst_hbm.at[wid], dst2)

    iota = lax.iota(jnp.int32, 16)
    row16 = [iota + g * 16 for g in range(K2 // 16)]

    def zero_acc():
        @pl.when(sid < NS - 1)
        def _():
            pltpu.sync_copy(zer_hbm.at[pl.ds(0, RPT)],
                            acc_sh.at[pl.ds(sid * RPT, RPT)])

        @pl.when(sid == NS - 1)
        def _():
            pltpu.sync_copy(zer_hbm.at[pl.ds(0, RPT_LAST)],
                            acc_sh.at[pl.ds((NS - 1) * RPT, RPT_LAST)])

    def dump_acc(dst):
        @pl.when(sid < NS - 1)
        def _():
            pltpu.sync_copy(acc_sh.at[pl.ds(sid * RPT, RPT)],
                            dst.at[cid, pl.ds(sid * RPT, RPT)])

        @pl.when(sid == NS - 1)
        def _():
            pltpu.sync_copy(acc_sh.at[pl.ds((NS - 1) * RPT, RPT_LAST)],
                            dst.at[cid, pl.ds((NS - 1) * RPT, RPT_LAST)])

    for chk in range(nch):
        zero_acc()
        pltpu.sync_copy(w_hbm.at[0, wid], w_v)
        plsc.subcore_barrier()

        def p2(k, c, _chk=chk):
            @pl.when(k == NKC2 // 2)
            def _():
                pltpu.sync_copy(w_hbm.at[1, wid], w_v)

            woff = jnp.where(k >= NKC2 // 2, k * K2 - WHALF, k * K2)
            pltpu.sync_copy(hl_hbm[_chk].at[src2.at[k]], rows_hl)
            w16s = [w_v[pl.ds(woff + g * 16, 16)] for g in range(K2 // 16)]

            def sbody(ch, c2):
                colc = jnp.full((16,), ch, jnp.int32)
                for g in range(K2 // 16):
                    v = plsc.load_gather(rows_hl, [row16[g], colc])
                    plsc.store_scatter(rows_hl, [row16[g], colc],
                                       v * w16s[g])
                return c2
            lax.fori_loop(0, CP, sbody, 0)
            pltpu.sync_copy(rows_hl, acc_sh.at[dst2.at[k]], add=True)
            return c
        lax.fori_loop(0, NKC2, p2, 0)
        plsc.subcore_barrier()
        dump_acc(num_hbm[chk])
        plsc.subcore_barrier()


@functools.cache
def _make_sc_wide(c):
    nch = max(1, c // CP)
    p1 = pl.kernel(
        functools.partial(_sc_wide_p1_body, nch),
        out_type=(jax.ShapeDtypeStruct((2, NW, WHALF), jnp.float32),
                  jax.ShapeDtypeStruct((NW, NP), jnp.float32)),
        mesh=_MESH,
        compiler_params=_SC_PARAMS,
        scratch_types=[
            pltpu.VMEM((K1, CP), jnp.float32),
            pltpu.VMEM((K1, CP), jnp.float32),
            pltpu.VMEM((NP,), jnp.float32),
            pltpu.VMEM((2, WHALF), jnp.float32),
            pltpu.VMEM((NP,), jnp.float32),
            pltpu.VMEM((NKC1, K1), jnp.int32),
            pltpu.VMEM((NKC1, K1), jnp.int32),
            pltpu.VMEM((nch * CP,), jnp.float32),
        ],
    )
    p2 = pl.kernel(
        functools.partial(_sc_wide_p2_body, nch),
        out_type=tuple([jax.ShapeDtypeStruct((2, NP, CP), jnp.float32)] * nch),
        mesh=_MESH,
        compiler_params=_SC_PARAMS,
        scratch_types=[
            pltpu.VMEM((K2, CP), jnp.float32),
            pltpu.VMEM((WHALF,), jnp.float32),
            pltpu.VMEM((NKC2, K2), jnp.int32),
            pltpu.VMEM((NKC2, K2), jnp.int32),
            pltpu.VMEM_SHARED((NP, CP), jnp.float32),
        ],
    )
    return p1, p2, nch


def _gat_wide(x, idx3, p, act="silu"):
    srcp3, dstp3, srcp3b, dstp3b = idx3
    c = p["Wl"].shape[1]
    p1, p2, nch = _make_sc_wide(c)
    projs, attp = _proj_wide(x, p, nch)
    hl = projs[:nch]
    hr = projs[nch:2 * nch]
    s_arr = projs[2 * nch]
    w, denparts = p1(*hl, *hr, s_arr, srcp3, dstp3, attp)
    zer = jnp.zeros((RPT, CP), jnp.float32)
    nums = p2(*hl, srcp3b, dstp3b, w, zer)
    return _fin_wide(list(nums), denparts, p["b"], c, act)


def _scatter_mean(srcv, index, num_segments):
    s = jax.ops.segment_sum(srcv, index, num_segments=num_segments)
    cnt = jax.ops.segment_sum(jnp.ones((srcv.shape[0],), srcv.dtype), index,
                              num_segments=num_segments)
    return s / jnp.clip(cnt, 1, None)[:, None]


def _pad_rows(x):
    return jnp.zeros((NP, x.shape[1]), jnp.float32).at[:N].set(x)


def kernel(el, lat, batch, edge_index, params):
    loops = jnp.arange(N, dtype=edge_index.dtype)
    src = jnp.concatenate([edge_index[0], loops])
    dst = jnp.concatenate([edge_index[1], loops])
    pad = jnp.full((E_PAD - E,), DUMMY, src.dtype)
    srcp = jnp.concatenate([src, pad])
    dstp = jnp.concatenate([dst, pad])
    idx3 = (srcp.reshape(NW, NKC1, K1), dstp.reshape(NW, NKC1, K1),
            srcp.reshape(NW, NKC2, K2), dstp.reshape(NW, NKC2, K2))

    x_el = _pad_rows(params["emb"][el])
    x_el = _gat_wide(x_el, idx3, params["element"])
    x_lat = _gat_wide(_pad_rows(lat), idx3, params["latent"])
    x = jnp.concatenate([x_el, x_lat], axis=-1)
    x = _gat_wide(x, idx3, params["pre"])
    for p in params["coords"][:4]:
        x = _gat_wide(x, idx3, p)

    tbl = _tbl_from_x(x.T, params["coords"][4])
    parts = _gat3_parts(tbl, srcp, dstp, params["coords"][4])
    coords_t = _finalize(parts, params["coords"][4]["b"], sig=True)
    coords = coords_t[:3, :N].T

    lengths = _chain3(coords_t, srcp, dstp,
                      [params["post_len"]] + list(params["len_blocks"]))
    lengths = _scatter_mean(lengths[:3, :N].T, batch, N_GRAPHS)
    angles = _chain3(coords_t, srcp, dstp,
                     [params["post_ang"]] + list(params["ang_blocks"]))
    angles = _scatter_mean(angles[:3, :N].T, batch, N_GRAPHS)
    return coords, lengths, angles


# speedup vs baseline: 3.8487x; 1.1024x over previous
import functools

import jax
import jax.numpy as jnp
from jax import lax
from jax.experimental import pallas as pl
from jax.experimental.pallas import tpu as pltpu
from jax.experimental.pallas import tpu_sc as plsc

N_GRAPHS = 100
N = 10000
NP = 10016
DUMMY = 10008
E = 330000
NW = 32
NS = 16
EPW = 10368
E_PAD = NW * EPW
CHUNK3 = 2592
CP = 128
K1 = 128
NKC1 = EPW // K1
K2 = 96
NKC2 = EPW // K2
WHALF = EPW // 2
RPT = 632
RPT_LAST = NP - (NS - 1) * RPT
NEG_SLOPE = 0.2


def _lrelu(z):
    return jnp.maximum(z, z * NEG_SLOPE)


def _silu(z):
    return z * jax.nn.sigmoid(z)


_MESH = plsc.VectorSubcoreMesh(core_axis_name="c", subcore_axis_name="s")
_SC_PARAMS = pltpu.CompilerParams(needs_layout_passes=False)
_TC_PARAMS = pltpu.CompilerParams(vmem_limit_bytes=60 * 1024 * 1024)


def _tbl_from_x_kernel(x_ref, wlt_ref, wrt_ref, bl_ref, br_ref, att_ref, tbl_ref):
    x = x_ref[...]
    hl = wlt_ref[...] @ x + bl_ref[...]
    hr = wrt_ref[...] @ x + br_ref[...]
    att = att_ref[...]
    s = jnp.sum(_lrelu(hl + hr) * att, axis=0, keepdims=True)
    e3 = (jax.lax.broadcasted_iota(jnp.int32, (4, 1), 0) == 3).astype(jnp.float32)
    hrs = hr * (1.0 - e3) + s * e3
    tbl_ref[...] = jnp.concatenate([hl, hrs], axis=0)


def _tbl_from_x(xt, p):
    cin = p["Wl"].shape[0]
    cin_p = xt.shape[0]
    wlt = jnp.zeros((4, cin_p), jnp.float32).at[:3, :cin].set(p["Wl"].T)
    wrt = jnp.zeros((4, cin_p), jnp.float32).at[:3, :cin].set(p["Wr"].T)
    bl = jnp.zeros((4, 1), jnp.float32).at[:3, 0].set(p["bl"])
    br = jnp.zeros((4, 1), jnp.float32).at[:3, 0].set(p["br"])
    att = jnp.zeros((4, 1), jnp.float32).at[:3, 0].set(p["att"])
    return pl.pallas_call(
        _tbl_from_x_kernel,
        out_shape=jax.ShapeDtypeStruct((8, NP), jnp.float32),
    )(xt, wlt, wrt, bl, br, att)


def _finalize_kernel(sig, parts_ref, b_ref, x_ref):
    summed = jnp.sum(parts_ref[...], axis=0)
    den = summed[3:4, :]
    h = summed / (den + 1e-16) + b_ref[...]
    x_ref[...] = jax.nn.sigmoid(h) if sig else _silu(h)


def _finalize(parts, b, sig=False):
    bp = jnp.zeros((4, 1), jnp.float32).at[:3, 0].set(b)
    return pl.pallas_call(
        functools.partial(_finalize_kernel, sig),
        out_shape=jax.ShapeDtypeStruct((4, NP), jnp.float32),
    )(parts, bp)


def _sc_edge3_body(tbl_hbm, src_hbm, dst_hbm, att_hbm, out_hbm,
                   tbl_v, acc_v, src_v, dst_v, att_v):
    wid = lax.axis_index("s") * 2 + lax.axis_index("c")

    def zbody(i, c):
        acc_v[pl.ds(i * 16, 16)] = jnp.zeros((16,), jnp.float32)
        return c
    lax.fori_loop(0, (NP * 4) // 16, zbody, 0)

    pltpu.sync_copy(tbl_hbm, tbl_v)
    pltpu.sync_copy(att_hbm, att_v)
    a0 = att_v[pl.ds(0, 16)]
    a1 = att_v[pl.ds(16, 16)]
    a2 = att_v[pl.ds(32, 16)]

    base = wid * EPW

    def group(g, c):
        s16 = src_v[pl.ds(g * 16, 16)]
        d16 = dst_v[pl.ds(g * 16, 16)]
        x0 = plsc.load_gather(tbl_v, [s16])
        x1 = plsc.load_gather(tbl_v, [s16 + NP])
        x2 = plsc.load_gather(tbl_v, [s16 + 2 * NP])
        y0 = plsc.load_gather(tbl_v, [d16 + 4 * NP])
        y1 = plsc.load_gather(tbl_v, [d16 + 5 * NP])
        y2 = plsc.load_gather(tbl_v, [d16 + 6 * NP])
        sv = plsc.load_gather(tbl_v, [d16 + 7 * NP])
        logit = (a0 * _lrelu(x0 + y0) + a1 * _lrelu(x1 + y1)
                 + a2 * _lrelu(x2 + y2))
        w = jnp.exp(logit - sv)
        plsc.addupdate_scatter(acc_v, [d16], w * x0)
        plsc.addupdate_scatter(acc_v, [d16 + NP], w * x1)
        plsc.addupdate_scatter(acc_v, [d16 + 2 * NP], w * x2)
        plsc.addupdate_scatter(acc_v, [d16 + 3 * NP], w)
        return c

    for k in range(EPW // CHUNK3):
        pltpu.sync_copy(src_hbm.at[pl.ds(base + k * CHUNK3, CHUNK3)], src_v)
        pltpu.sync_copy(dst_hbm.at[pl.ds(base + k * CHUNK3, CHUNK3)], dst_v)
        lax.fori_loop(0, CHUNK3 // 16, group, 0)

    pltpu.sync_copy(acc_v, out_hbm.at[wid])


_sc_edge3 = functools.partial(
    pl.kernel,
    out_type=jax.ShapeDtypeStruct((NW, NP * 4), jnp.float32),
    mesh=_MESH,
    compiler_params=_SC_PARAMS,
    scratch_types=[
        pltpu.VMEM((NP * 8,), jnp.float32),
        pltpu.VMEM((NP * 4,), jnp.float32),
        pltpu.VMEM((CHUNK3,), jnp.int32),
        pltpu.VMEM((CHUNK3,), jnp.int32),
        pltpu.VMEM((48,), jnp.float32),
    ],
)(_sc_edge3_body)


def _gat3_parts(tbl, srcp, dstp, p):
    att48 = jnp.repeat(p["att"].astype(jnp.float32), 16)
    parts = _sc_edge3(tbl.reshape(-1), srcp, dstp, att48)
    return parts.reshape(NW, 4, NP)


def _chain3(xt, srcp, dstp, plist):
    tbl = _tbl_from_x(xt, plist[0])
    for i, p in enumerate(plist):
        parts = _gat3_parts(tbl, srcp, dstp, p)
        if i + 1 < len(plist):
            xnext = _finalize(parts, p["b"])
            tbl = _tbl_from_x(xnext, plist[i + 1])
        else:
            return _finalize(parts, p["b"])


def _proj_wide_body(nch, x_ref, wl_ref, bl_ref, wr_ref, br_ref, att_ref,
                    *out_refs):
    x = x_ref[...]
    hl = x @ wl_ref[...] + bl_ref[...]
    hr = x @ wr_ref[...] + br_ref[...]
    s = jnp.sum(_lrelu(hl + hr) * att_ref[...], axis=-1, keepdims=True)
    for i in range(nch):
        out_refs[i][...] = hl[:, i * CP:(i + 1) * CP]
        out_refs[nch + i][...] = hr[:, i * CP:(i + 1) * CP]
    out_refs[2 * nch][...] = s[:, 0]


def _proj_wide(x, p, nch):
    cin = p["Wl"].shape[0]
    c = p["Wl"].shape[1]
    cpad = nch * CP
    wl = jnp.zeros((cin, cpad), jnp.float32).at[:, :c].set(p["Wl"])
    wr = jnp.zeros((cin, cpad), jnp.float32).at[:, :c].set(p["Wr"])
    bl = jnp.zeros((1, cpad), jnp.float32).at[0, :c].set(p["bl"])
    br = jnp.zeros((1, cpad), jnp.float32).at[0, :c].set(p["br"])
    att = jnp.zeros((1, cpad), jnp.float32).at[0, :c].set(p["att"])
    outs = ([jax.ShapeDtypeStruct((NP, CP), jnp.float32)] * (2 * nch)
            + [jax.ShapeDtypeStruct((NP,), jnp.float32)])
    return pl.pallas_call(
        functools.partial(_proj_wide_body, nch),
        out_shape=tuple(outs),
        compiler_params=_TC_PARAMS,
    )(x, wl, bl, wr, br, att), att[0]


def _fin_wide_body(nch, c, act, *refs):
    num_refs = refs[:nch]
    den_ref, b_ref = refs[nch], refs[nch + 1]
    x_ref = refs[nch + 2]
    num = jnp.concatenate([r[...][0] + r[...][1] for r in num_refs], axis=-1)
    den = jnp.sum(den_ref[...], axis=0)[:, None]
    h = num[:, :c] / (den + 1e-16) + b_ref[...]
    x_ref[...] = _silu(h) if act == "silu" else h


def _fin_wide(numparts, denparts, b, c, act="silu"):
    return pl.pallas_call(
        functools.partial(_fin_wide_body, len(numparts), c, act),
        out_shape=jax.ShapeDtypeStruct((NP, c), jnp.float32),
        compiler_params=_TC_PARAMS,
    )(*numparts, denparts, b[None, :])


def _sc_wide_p1_body(nch, *refs):
    hl_hbm = refs[:nch]
    hr_hbm = refs[nch:2 * nch]
    s_hbm, src_hbm, dst_hbm, att_hbm = refs[2 * nch:2 * nch + 4]
    w_hbm, den_hbm = refs[2 * nch + 4:2 * nch + 6]
    (bhl0, bhr0, bhl1, bhr1, s_all, w_v, den_v, src2, dst2,
     att_v, sem0l, sem0r, sem1l, sem1r) = refs[2 * nch + 6:]

    cid = lax.axis_index("c")
    sid = lax.axis_index("s")
    wid = sid * 2 + cid

    def zden(i, c):
        den_v[pl.ds(i * 16, 16)] = jnp.zeros((16,), jnp.float32)
        return c
    lax.fori_loop(0, NP // 16, zden, 0)
    pltpu.sync_copy(att_hbm, att_v)
    pltpu.sync_copy(s_hbm, s_all)
    pltpu.sync_copy(src_hbm.at[wid], src2)
    pltpu.sync_copy(dst_hbm.at[wid], dst2)

    iota = lax.iota(jnp.int32, 16)
    row16 = [iota + g * 16 for g in range(K1 // 16)]

    slots = [(0, bhl0, bhr0, sem0l, sem0r),
             (nch - 1, bhl1, bhr1, sem1l, sem1r)]

    def start(slot, k):
        i, bhl, bhr, sl, sr = slots[slot]
        pltpu.make_async_copy(hl_hbm[i].at[src2.at[k]], bhl, sl).start()
        pltpu.make_async_copy(hr_hbm[i].at[dst2.at[k]], bhr, sr).start()

    def wait(slot, k):
        i, bhl, bhr, sl, sr = slots[slot]
        pltpu.make_async_copy(hl_hbm[i].at[src2.at[k]], bhl, sl).wait()
        pltpu.make_async_copy(hr_hbm[i].at[dst2.at[k]], bhr, sr).wait()

    def logit_part(slot, carry):
        i, bhl, bhr, _, _ = slots[slot]

        def cbody(ch, carry, _i=i, _bhl=bhl, _bhr=bhr):
            colc = jnp.full((16,), ch, jnp.int32)
            aspl = plsc.load_gather(att_v, [colc + _i * CP])
            out = []
            for g in range(K1 // 16):
                a = plsc.load_gather(_bhl, [row16[g], colc])
                b = plsc.load_gather(_bhr, [row16[g], colc])
                l = _lrelu(a + b)
                out.append(carry[g] + aspl * l)
            return tuple(out)
        return lax.fori_loop(0, CP, cbody, carry)

    def wden(k, carry):
        for g in range(K1 // 16):
            d16 = dst2[k, pl.ds(g * 16, 16)]
            sg = plsc.load_gather(s_all, [d16])
            w16 = jnp.exp(carry[g] - sg)
            plsc.addupdate_scatter(den_v, [d16], w16)
            idx = k * K1 + g * 16
            h = idx // WHALF
            w_v[h, pl.ds(idx - h * WHALF, 16)] = w16

    zcarry = tuple(jnp.zeros((16,), jnp.float32) for _ in range(K1 // 16))

    if nch == 1:
        start(0, 0)
        start(1, 1)

        def pair(j, c):
            a = 2 * j
            wait(0, a)
            carry = logit_part(0, zcarry)
            start(0, a + 2)
            wden(a, carry)
            wait(1, a + 1)
            carry = logit_part(1, zcarry)
            nxt = jnp.minimum(a + 3, NKC1 - 1)
            start(1, nxt)
            wden(a + 1, carry)
            return c
        lax.fori_loop(0, (NKC1 - 1) // 2, pair, 0)
        wait(0, NKC1 - 1)
        carry = logit_part(0, zcarry)
        wden(NKC1 - 1, carry)
        wait(1, NKC1 - 1)
    else:
        start(0, 0)
        start(1, 0)

        def chunk(k, c):
            nxt = jnp.minimum(k + 1, NKC1 - 1)
            wait(0, k)
            carry = logit_part(0, zcarry)
            start(0, nxt)
            wait(1, k)
            carry = logit_part(1, carry)
            start(1, nxt)
            wden(k, carry)
            return c
        lax.fori_loop(0, NKC1, chunk, 0)
        wait(0, NKC1 - 1)
        wait(1, NKC1 - 1)

    pltpu.sync_copy(w_v.at[0], w_hbm.at[0, wid])
    pltpu.sync_copy(w_v.at[1], w_hbm.at[1, wid])
    pltpu.sync_copy(den_v, den_hbm.at[wid])


def _sc_wide_p2_body(nch, *refs):
    hl_hbm = refs[:nch]
    src_hbm, dst_hbm, w_hbm, zer_hbm = refs[nch:nch + 4]
    num_hbm = refs[nch + 4:2 * nch + 4]
    rows_hl, w_v, src2, dst2, acc_sh = refs[2 * nch + 4:]

    cid = lax.axis_index("c")
    sid = lax.axis_index("s")
    wid = sid * 2 + cid

    pltpu.sync_copy(src_hbm.at[wid], src2)
    pltpu.sync_copy(dst_hbm.at[wid], dst2)

    iota = lax.iota(jnp.int32, 16)
    row16 = [iota + g * 16 for g in range(K2 // 16)]

    def zero_acc():
        @pl.when(sid < NS - 1)
        def _():
            pltpu.sync_copy(zer_hbm.at[pl.ds(0, RPT)],
                            acc_sh.at[pl.ds(sid * RPT, RPT)])

        @pl.when(sid == NS - 1)
        def _():
            pltpu.sync_copy(zer_hbm.at[pl.ds(0, RPT_LAST)],
                            acc_sh.at[pl.ds((NS - 1) * RPT, RPT_LAST)])

    def dump_acc(dst):
        @pl.when(sid < NS - 1)
        def _():
            pltpu.sync_copy(acc_sh.at[pl.ds(sid * RPT, RPT)],
                            dst.at[cid, pl.ds(sid * RPT, RPT)])

        @pl.when(sid == NS - 1)
        def _():
            pltpu.sync_copy(acc_sh.at[pl.ds((NS - 1) * RPT, RPT_LAST)],
                            dst.at[cid, pl.ds((NS - 1) * RPT, RPT_LAST)])

    for chk in range(nch):
        zero_acc()
        pltpu.sync_copy(w_hbm.at[0, wid], w_v)
        plsc.subcore_barrier()

        def p2(k, c, _chk=chk):
            @pl.when(k == NKC2 // 2)
            def _():
                pltpu.sync_copy(w_hbm.at[1, wid], w_v)

            woff = jnp.where(k >= NKC2 // 2, k * K2 - WHALF, k * K2)
            pltpu.sync_copy(hl_hbm[_chk].at[src2.at[k]], rows_hl)
            w16s = [w_v[pl.ds(woff + g * 16, 16)] for g in range(K2 // 16)]

            def sbody(ch, c2):
                colc = jnp.full((16,), ch, jnp.int32)
                for g in range(K2 // 16):
                    v = plsc.load_gather(rows_hl, [row16[g], colc])
                    plsc.store_scatter(rows_hl, [row16[g], colc],
                                       v * w16s[g])
                return c2
            lax.fori_loop(0, CP, sbody, 0)
            pltpu.sync_copy(rows_hl, acc_sh.at[dst2.at[k]], add=True)
            return c
        lax.fori_loop(0, NKC2, p2, 0)
        plsc.subcore_barrier()
        dump_acc(num_hbm[chk])
        plsc.subcore_barrier()


@functools.cache
def _make_sc_wide(c):
    nch = max(1, c // CP)
    p1 = pl.kernel(
        functools.partial(_sc_wide_p1_body, nch),
        out_type=(jax.ShapeDtypeStruct((2, NW, WHALF), jnp.float32),
                  jax.ShapeDtypeStruct((NW, NP), jnp.float32)),
        mesh=_MESH,
        compiler_params=_SC_PARAMS,
        scratch_types=[
            pltpu.VMEM((K1, CP), jnp.float32),
            pltpu.VMEM((K1, CP), jnp.float32),
            pltpu.VMEM((K1, CP), jnp.float32),
            pltpu.VMEM((K1, CP), jnp.float32),
            pltpu.VMEM((NP,), jnp.float32),
            pltpu.VMEM((2, WHALF), jnp.float32),
            pltpu.VMEM((NP,), jnp.float32),
            pltpu.VMEM((NKC1, K1), jnp.int32),
            pltpu.VMEM((NKC1, K1), jnp.int32),
            pltpu.VMEM((nch * CP,), jnp.float32),
            pltpu.SemaphoreType.DMA,
            pltpu.SemaphoreType.DMA,
            pltpu.SemaphoreType.DMA,
            pltpu.SemaphoreType.DMA,
        ],
    )
    p2 = pl.kernel(
        functools.partial(_sc_wide_p2_body, nch),
        out_type=tuple([jax.ShapeDtypeStruct((2, NP, CP), jnp.float32)] * nch),
        mesh=_MESH,
        compiler_params=_SC_PARAMS,
        scratch_types=[
            pltpu.VMEM((K2, CP), jnp.float32),
            pltpu.VMEM((WHALF,), jnp.float32),
            pltpu.VMEM((NKC2, K2), jnp.int32),
            pltpu.VMEM((NKC2, K2), jnp.int32),
            pltpu.VMEM_SHARED((NP, CP), jnp.float32),
        ],
    )
    return p1, p2, nch


def _gat_wide(x, idx3, p, act="silu"):
    srcp3, dstp3, srcp3b, dstp3b = idx3
    c = p["Wl"].shape[1]
    p1, p2, nch = _make_sc_wide(c)
    projs, attp = _proj_wide(x, p, nch)
    hl = projs[:nch]
    hr = projs[nch:2 * nch]
    s_arr = projs[2 * nch]
    w, denparts = p1(*hl, *hr, s_arr, srcp3, dstp3, attp)
    zer = jnp.zeros((RPT, CP), jnp.float32)
    nums = p2(*hl, srcp3b, dstp3b, w, zer)
    return _fin_wide(list(nums), denparts, p["b"], c, act)


def _scatter_mean(srcv, index, num_segments):
    s = jax.ops.segment_sum(srcv, index, num_segments=num_segments)
    cnt = jax.ops.segment_sum(jnp.ones((srcv.shape[0],), srcv.dtype), index,
                              num_segments=num_segments)
    return s / jnp.clip(cnt, 1, None)[:, None]


def _pad_rows(x):
    return jnp.zeros((NP, x.shape[1]), jnp.float32).at[:N].set(x)


def kernel(el, lat, batch, edge_index, params):
    loops = jnp.arange(N, dtype=edge_index.dtype)
    src = jnp.concatenate([edge_index[0], loops])
    dst = jnp.concatenate([edge_index[1], loops])
    pad = jnp.full((E_PAD - E,), DUMMY, src.dtype)
    srcp = jnp.concatenate([src, pad])
    dstp = jnp.concatenate([dst, pad])
    idx3 = (srcp.reshape(NW, NKC1, K1), dstp.reshape(NW, NKC1, K1),
            srcp.reshape(NW, NKC2, K2), dstp.reshape(NW, NKC2, K2))

    x_el = _pad_rows(params["emb"][el])
    x_el = _gat_wide(x_el, idx3, params["element"])
    x_lat = _gat_wide(_pad_rows(lat), idx3, params["latent"])
    x = jnp.concatenate([x_el, x_lat], axis=-1)
    x = _gat_wide(x, idx3, params["pre"])
    for p in params["coords"][:4]:
        x = _gat_wide(x, idx3, p)

    tbl = _tbl_from_x(x.T, params["coords"][4])
    parts = _gat3_parts(tbl, srcp, dstp, params["coords"][4])
    coords_t = _finalize(parts, params["coords"][4]["b"], sig=True)
    coords = coords_t[:3, :N].T

    lengths = _chain3(coords_t, srcp, dstp,
                      [params["post_len"]] + list(params["len_blocks"]))
    lengths = _scatter_mean(lengths[:3, :N].T, batch, N_GRAPHS)
    angles = _chain3(coords_t, srcp, dstp,
                     [params["post_ang"]] + list(params["ang_blocks"]))
    angles = _scatter_mean(angles[:3, :N].T, batch, N_GRAPHS)
    return coords, lengths, angles


# speedup vs baseline: 3.9511x; 1.0266x over previous
import functools

import jax
import jax.numpy as jnp
from jax import lax
from jax.experimental import pallas as pl
from jax.experimental.pallas import tpu as pltpu
from jax.experimental.pallas import tpu_sc as plsc

N_GRAPHS = 100
N = 10000
NP = 10016
DUMMY = 10008
E = 330000
NW = 32
NS = 16
EPW = 10368
E_PAD = NW * EPW
CHUNK3 = 2592
CP = 128
K1 = 128
NKC1 = EPW // K1
K2 = 96
NKC2 = EPW // K2
WHALF = EPW // 2
RPT = 632
RPT_LAST = NP - (NS - 1) * RPT
NEG_SLOPE = 0.2


def _lrelu(z):
    return jnp.maximum(z, z * NEG_SLOPE)


def _silu(z):
    return z * jax.nn.sigmoid(z)


_MESH = plsc.VectorSubcoreMesh(core_axis_name="c", subcore_axis_name="s")
_SC_PARAMS = pltpu.CompilerParams(needs_layout_passes=False)
_TC_PARAMS = pltpu.CompilerParams(vmem_limit_bytes=60 * 1024 * 1024)


def _tbl_from_x_kernel(x_ref, wlt_ref, wrt_ref, bl_ref, br_ref, att_ref, tbl_ref):
    x = x_ref[...]
    hl = wlt_ref[...] @ x + bl_ref[...]
    hr = wrt_ref[...] @ x + br_ref[...]
    att = att_ref[...]
    s = jnp.sum(_lrelu(hl + hr) * att, axis=0, keepdims=True)
    e3 = (jax.lax.broadcasted_iota(jnp.int32, (4, 1), 0) == 3).astype(jnp.float32)
    hrs = hr * (1.0 - e3) + s * e3
    tbl_ref[...] = jnp.concatenate([hl, hrs], axis=0)


def _tbl_from_x(xt, p):
    cin = p["Wl"].shape[0]
    cin_p = xt.shape[0]
    wlt = jnp.zeros((4, cin_p), jnp.float32).at[:3, :cin].set(p["Wl"].T)
    wrt = jnp.zeros((4, cin_p), jnp.float32).at[:3, :cin].set(p["Wr"].T)
    bl = jnp.zeros((4, 1), jnp.float32).at[:3, 0].set(p["bl"])
    br = jnp.zeros((4, 1), jnp.float32).at[:3, 0].set(p["br"])
    att = jnp.zeros((4, 1), jnp.float32).at[:3, 0].set(p["att"])
    return pl.pallas_call(
        _tbl_from_x_kernel,
        out_shape=jax.ShapeDtypeStruct((8, NP), jnp.float32),
    )(xt, wlt, wrt, bl, br, att)


def _finalize_kernel(sig, parts_ref, b_ref, x_ref):
    summed = jnp.sum(parts_ref[...], axis=0)
    den = summed[3:4, :]
    h = summed / (den + 1e-16) + b_ref[...]
    x_ref[...] = jax.nn.sigmoid(h) if sig else _silu(h)


def _finalize(parts, b, sig=False):
    bp = jnp.zeros((4, 1), jnp.float32).at[:3, 0].set(b)
    return pl.pallas_call(
        functools.partial(_finalize_kernel, sig),
        out_shape=jax.ShapeDtypeStruct((4, NP), jnp.float32),
    )(parts, bp)


def _sc_edge3_body(tbl_hbm, src_hbm, dst_hbm, att_hbm, out_hbm,
                   tbl_v, acc_v, src_v, dst_v, att_v):
    wid = lax.axis_index("s") * 2 + lax.axis_index("c")

    def zbody(i, c):
        acc_v[pl.ds(i * 16, 16)] = jnp.zeros((16,), jnp.float32)
        return c
    lax.fori_loop(0, (NP * 4) // 16, zbody, 0)

    pltpu.sync_copy(tbl_hbm, tbl_v)
    pltpu.sync_copy(att_hbm, att_v)
    a0 = att_v[pl.ds(0, 16)]
    a1 = att_v[pl.ds(16, 16)]
    a2 = att_v[pl.ds(32, 16)]

    base = wid * EPW

    def group(g2, c):
        for u in range(2):
            g = g2 * 2 + u
            s16 = src_v[pl.ds(g * 16, 16)]
            d16 = dst_v[pl.ds(g * 16, 16)]
            x0 = plsc.load_gather(tbl_v, [s16])
            x1 = plsc.load_gather(tbl_v, [s16 + NP])
            x2 = plsc.load_gather(tbl_v, [s16 + 2 * NP])
            y0 = plsc.load_gather(tbl_v, [d16 + 4 * NP])
            y1 = plsc.load_gather(tbl_v, [d16 + 5 * NP])
            y2 = plsc.load_gather(tbl_v, [d16 + 6 * NP])
            sv = plsc.load_gather(tbl_v, [d16 + 7 * NP])
            logit = (a0 * _lrelu(x0 + y0) + a1 * _lrelu(x1 + y1)
                     + a2 * _lrelu(x2 + y2))
            w = jnp.exp(logit - sv)
            plsc.addupdate_scatter(acc_v, [d16], w * x0)
            plsc.addupdate_scatter(acc_v, [d16 + NP], w * x1)
            plsc.addupdate_scatter(acc_v, [d16 + 2 * NP], w * x2)
            plsc.addupdate_scatter(acc_v, [d16 + 3 * NP], w)
        return c

    for k in range(EPW // CHUNK3):
        pltpu.sync_copy(src_hbm.at[pl.ds(base + k * CHUNK3, CHUNK3)], src_v)
        pltpu.sync_copy(dst_hbm.at[pl.ds(base + k * CHUNK3, CHUNK3)], dst_v)
        lax.fori_loop(0, CHUNK3 // 32, group, 0)

    pltpu.sync_copy(acc_v, out_hbm.at[wid])


_sc_edge3 = functools.partial(
    pl.kernel,
    out_type=jax.ShapeDtypeStruct((NW, NP * 4), jnp.float32),
    mesh=_MESH,
    compiler_params=_SC_PARAMS,
    scratch_types=[
        pltpu.VMEM((NP * 8,), jnp.float32),
        pltpu.VMEM((NP * 4,), jnp.float32),
        pltpu.VMEM((CHUNK3,), jnp.int32),
        pltpu.VMEM((CHUNK3,), jnp.int32),
        pltpu.VMEM((48,), jnp.float32),
    ],
)(_sc_edge3_body)


def _gat3_parts(tbl, srcp, dstp, p):
    att48 = jnp.repeat(p["att"].astype(jnp.float32), 16)
    parts = _sc_edge3(tbl.reshape(-1), srcp, dstp, att48)
    return parts.reshape(NW, 4, NP)


def _chain3(xt, srcp, dstp, plist):
    tbl = _tbl_from_x(xt, plist[0])
    for i, p in enumerate(plist):
        parts = _gat3_parts(tbl, srcp, dstp, p)
        if i + 1 < len(plist):
            xnext = _finalize(parts, p["b"])
            tbl = _tbl_from_x(xnext, plist[i + 1])
        else:
            return _finalize(parts, p["b"])


def _proj_wide_body(nch, x_ref, wl_ref, bl_ref, wr_ref, br_ref, att_ref,
                    *out_refs):
    x = x_ref[...]
    hl = x @ wl_ref[...] + bl_ref[...]
    hr = x @ wr_ref[...] + br_ref[...]
    s = jnp.sum(_lrelu(hl + hr) * att_ref[...], axis=-1, keepdims=True)
    for i in range(nch):
        out_refs[i][...] = hl[:, i * CP:(i + 1) * CP]
        out_refs[nch + i][...] = hr[:, i * CP:(i + 1) * CP]
    out_refs[2 * nch][...] = s[:, 0]


def _proj_wide(x, p, nch):
    cin = p["Wl"].shape[0]
    c = p["Wl"].shape[1]
    cpad = nch * CP
    wl = jnp.zeros((cin, cpad), jnp.float32).at[:, :c].set(p["Wl"])
    wr = jnp.zeros((cin, cpad), jnp.float32).at[:, :c].set(p["Wr"])
    bl = jnp.zeros((1, cpad), jnp.float32).at[0, :c].set(p["bl"])
    br = jnp.zeros((1, cpad), jnp.float32).at[0, :c].set(p["br"])
    att = jnp.zeros((1, cpad), jnp.float32).at[0, :c].set(p["att"])
    outs = ([jax.ShapeDtypeStruct((NP, CP), jnp.float32)] * (2 * nch)
            + [jax.ShapeDtypeStruct((NP,), jnp.float32)])
    return pl.pallas_call(
        functools.partial(_proj_wide_body, nch),
        out_shape=tuple(outs),
        compiler_params=_TC_PARAMS,
    )(x, wl, bl, wr, br, att), att[0]


def _fin_wide_body(nch, c, act, *refs):
    num_refs = refs[:nch]
    den_ref, b_ref = refs[nch], refs[nch + 1]
    x_ref = refs[nch + 2]
    num = jnp.concatenate([r[...][0] + r[...][1] for r in num_refs], axis=-1)
    den = jnp.sum(den_ref[...], axis=0)[:, None]
    h = num[:, :c] / (den + 1e-16) + b_ref[...]
    x_ref[...] = _silu(h) if act == "silu" else h


def _fin_wide(numparts, denparts, b, c, act="silu"):
    return pl.pallas_call(
        functools.partial(_fin_wide_body, len(numparts), c, act),
        out_shape=jax.ShapeDtypeStruct((NP, c), jnp.float32),
        compiler_params=_TC_PARAMS,
    )(*numparts, denparts, b[None, :])


def _sc_wide_p1_body(nch, *refs):
    hl_hbm = refs[:nch]
    hr_hbm = refs[nch:2 * nch]
    s_hbm, src_hbm, dst_hbm, att_hbm = refs[2 * nch:2 * nch + 4]
    w_hbm, den_hbm = refs[2 * nch + 4:2 * nch + 6]
    (bhl0, bhr0, bhl1, bhr1, s_all, w_v, den_v, src2, dst2,
     att_v, sem0l, sem0r, sem1l, sem1r) = refs[2 * nch + 6:]

    cid = lax.axis_index("c")
    sid = lax.axis_index("s")
    wid = sid * 2 + cid

    def zden(i, c):
        den_v[pl.ds(i * 16, 16)] = jnp.zeros((16,), jnp.float32)
        return c
    lax.fori_loop(0, NP // 16, zden, 0)
    pltpu.sync_copy(att_hbm, att_v)
    pltpu.sync_copy(s_hbm, s_all)
    pltpu.sync_copy(src_hbm.at[wid], src2)
    pltpu.sync_copy(dst_hbm.at[wid], dst2)

    iota = lax.iota(jnp.int32, 16)
    row16 = [iota + g * 16 for g in range(K1 // 16)]

    slots = [(0, bhl0, bhr0, sem0l, sem0r),
             (nch - 1, bhl1, bhr1, sem1l, sem1r)]

    def start(slot, k):
        i, bhl, bhr, sl, sr = slots[slot]
        pltpu.make_async_copy(hl_hbm[i].at[src2.at[k]], bhl, sl).start()
        pltpu.make_async_copy(hr_hbm[i].at[dst2.at[k]], bhr, sr).start()

    def wait(slot, k):
        i, bhl, bhr, sl, sr = slots[slot]
        pltpu.make_async_copy(hl_hbm[i].at[src2.at[k]], bhl, sl).wait()
        pltpu.make_async_copy(hr_hbm[i].at[dst2.at[k]], bhr, sr).wait()

    def logit_part(slot, carry):
        i, bhl, bhr, _, _ = slots[slot]

        def cbody(c2, carry, _i=i, _bhl=bhl, _bhr=bhr):
            out = list(carry)
            for u in range(2):
                colc = jnp.full((16,), c2 * 2 + u, jnp.int32)
                aspl = plsc.load_gather(att_v, [colc + _i * CP])
                for g in range(K1 // 16):
                    a = plsc.load_gather(_bhl, [row16[g], colc])
                    b = plsc.load_gather(_bhr, [row16[g], colc])
                    l = _lrelu(a + b)
                    out[g] = out[g] + aspl * l
            return tuple(out)
        return lax.fori_loop(0, CP // 2, cbody, carry)

    def wden(k, carry):
        for g in range(K1 // 16):
            d16 = dst2[k, pl.ds(g * 16, 16)]
            sg = plsc.load_gather(s_all, [d16])
            w16 = jnp.exp(carry[g] - sg)
            plsc.addupdate_scatter(den_v, [d16], w16)
            idx = k * K1 + g * 16
            h = idx // WHALF
            w_v[h, pl.ds(idx - h * WHALF, 16)] = w16

    zcarry = tuple(jnp.zeros((16,), jnp.float32) for _ in range(K1 // 16))

    if nch == 1:
        start(0, 0)
        start(1, 1)

        def pair(j, c):
            a = 2 * j
            wait(0, a)
            carry = logit_part(0, zcarry)
            start(0, a + 2)
            wden(a, carry)
            wait(1, a + 1)
            carry = logit_part(1, zcarry)
            nxt = jnp.minimum(a + 3, NKC1 - 1)
            start(1, nxt)
            wden(a + 1, carry)
            return c
        lax.fori_loop(0, (NKC1 - 1) // 2, pair, 0)
        wait(0, NKC1 - 1)
        carry = logit_part(0, zcarry)
        wden(NKC1 - 1, carry)
        wait(1, NKC1 - 1)
    else:
        start(0, 0)
        start(1, 0)

        def chunk(k, c):
            nxt = jnp.minimum(k + 1, NKC1 - 1)
            wait(0, k)
            carry = logit_part(0, zcarry)
            start(0, nxt)
            wait(1, k)
            carry = logit_part(1, carry)
            start(1, nxt)
            wden(k, carry)
            return c
        lax.fori_loop(0, NKC1, chunk, 0)
        wait(0, NKC1 - 1)
        wait(1, NKC1 - 1)

    pltpu.sync_copy(w_v.at[0], w_hbm.at[0, wid])
    pltpu.sync_copy(w_v.at[1], w_hbm.at[1, wid])
    pltpu.sync_copy(den_v, den_hbm.at[wid])


def _sc_wide_p2_body(nch, *refs):
    hl_hbm = refs[:nch]
    src_hbm, dst_hbm, w_hbm, zer_hbm = refs[nch:nch + 4]
    num_hbm = refs[nch + 4:2 * nch + 4]
    rows_hl, w_v, src2, dst2, acc_sh = refs[2 * nch + 4:]

    cid = lax.axis_index("c")
    sid = lax.axis_index("s")
    wid = sid * 2 + cid

    pltpu.sync_copy(src_hbm.at[wid], src2)
    pltpu.sync_copy(dst_hbm.at[wid], dst2)

    iota = lax.iota(jnp.int32, 16)
    row16 = [iota + g * 16 for g in range(K2 // 16)]

    def zero_acc():
        @pl.when(sid < NS - 1)
        def _():
            pltpu.sync_copy(zer_hbm.at[pl.ds(0, RPT)],
                            acc_sh.at[pl.ds(sid * RPT, RPT)])

        @pl.when(sid == NS - 1)
        def _():
            pltpu.sync_copy(zer_hbm.at[pl.ds(0, RPT_LAST)],
                            acc_sh.at[pl.ds((NS - 1) * RPT, RPT_LAST)])

    def dump_acc(dst):
        @pl.when(sid < NS - 1)
        def _():
            pltpu.sync_copy(acc_sh.at[pl.ds(sid * RPT, RPT)],
                            dst.at[cid, pl.ds(sid * RPT, RPT)])

        @pl.when(sid == NS - 1)
        def _():
            pltpu.sync_copy(acc_sh.at[pl.ds((NS - 1) * RPT, RPT_LAST)],
                            dst.at[cid, pl.ds((NS - 1) * RPT, RPT_LAST)])

    for chk in range(nch):
        zero_acc()
        pltpu.sync_copy(w_hbm.at[0, wid], w_v)
        plsc.subcore_barrier()

        def p2(k, c, _chk=chk):
            @pl.when(k == NKC2 // 2)
            def _():
                pltpu.sync_copy(w_hbm.at[1, wid], w_v)

            woff = jnp.where(k >= NKC2 // 2, k * K2 - WHALF, k * K2)
            pltpu.sync_copy(hl_hbm[_chk].at[src2.at[k]], rows_hl)
            w16s = [w_v[pl.ds(woff + g * 16, 16)] for g in range(K2 // 16)]

            def sbody(ch2, c2):
                for u in range(2):
                    colc = jnp.full((16,), ch2 * 2 + u, jnp.int32)
                    for g in range(K2 // 16):
                        v = plsc.load_gather(rows_hl, [row16[g], colc])
                        plsc.store_scatter(rows_hl, [row16[g], colc],
                                           v * w16s[g])
                return c2
            lax.fori_loop(0, CP // 2, sbody, 0)
            pltpu.sync_copy(rows_hl, acc_sh.at[dst2.at[k]], add=True)
            return c
        lax.fori_loop(0, NKC2, p2, 0)
        plsc.subcore_barrier()
        dump_acc(num_hbm[chk])
        plsc.subcore_barrier()


@functools.cache
def _make_sc_wide(c):
    nch = max(1, c // CP)
    p1 = pl.kernel(
        functools.partial(_sc_wide_p1_body, nch),
        out_type=(jax.ShapeDtypeStruct((2, NW, WHALF), jnp.float32),
                  jax.ShapeDtypeStruct((NW, NP), jnp.float32)),
        mesh=_MESH,
        compiler_params=_SC_PARAMS,
        scratch_types=[
            pltpu.VMEM((K1, CP), jnp.float32),
            pltpu.VMEM((K1, CP), jnp.float32),
            pltpu.VMEM((K1, CP), jnp.float32),
            pltpu.VMEM((K1, CP), jnp.float32),
            pltpu.VMEM((NP,), jnp.float32),
            pltpu.VMEM((2, WHALF), jnp.float32),
            pltpu.VMEM((NP,), jnp.float32),
            pltpu.VMEM((NKC1, K1), jnp.int32),
            pltpu.VMEM((NKC1, K1), jnp.int32),
            pltpu.VMEM((nch * CP,), jnp.float32),
            pltpu.SemaphoreType.DMA,
            pltpu.SemaphoreType.DMA,
            pltpu.SemaphoreType.DMA,
            pltpu.SemaphoreType.DMA,
        ],
    )
    p2 = pl.kernel(
        functools.partial(_sc_wide_p2_body, nch),
        out_type=tuple([jax.ShapeDtypeStruct((2, NP, CP), jnp.float32)] * nch),
        mesh=_MESH,
        compiler_params=_SC_PARAMS,
        scratch_types=[
            pltpu.VMEM((K2, CP), jnp.float32),
            pltpu.VMEM((WHALF,), jnp.float32),
            pltpu.VMEM((NKC2, K2), jnp.int32),
            pltpu.VMEM((NKC2, K2), jnp.int32),
            pltpu.VMEM_SHARED((NP, CP), jnp.float32),
        ],
    )
    return p1, p2, nch


def _gat_wide(x, idx3, p, act="silu"):
    srcp3, dstp3, srcp3b, dstp3b = idx3
    c = p["Wl"].shape[1]
    p1, p2, nch = _make_sc_wide(c)
    projs, attp = _proj_wide(x, p, nch)
    hl = projs[:nch]
    hr = projs[nch:2 * nch]
    s_arr = projs[2 * nch]
    w, denparts = p1(*hl, *hr, s_arr, srcp3, dstp3, attp)
    zer = jnp.zeros((RPT, CP), jnp.float32)
    nums = p2(*hl, srcp3b, dstp3b, w, zer)
    return _fin_wide(list(nums), denparts, p["b"], c, act)


def _scatter_mean(srcv, index, num_segments):
    s = jax.ops.segment_sum(srcv, index, num_segments=num_segments)
    cnt = jax.ops.segment_sum(jnp.ones((srcv.shape[0],), srcv.dtype), index,
                              num_segments=num_segments)
    return s / jnp.clip(cnt, 1, None)[:, None]


def _pad_rows(x):
    return jnp.zeros((NP, x.shape[1]), jnp.float32).at[:N].set(x)


def kernel(el, lat, batch, edge_index, params):
    loops = jnp.arange(N, dtype=edge_index.dtype)
    src = jnp.concatenate([edge_index[0], loops])
    dst = jnp.concatenate([edge_index[1], loops])
    pad = jnp.full((E_PAD - E,), DUMMY, src.dtype)
    srcp = jnp.concatenate([src, pad])
    dstp = jnp.concatenate([dst, pad])
    idx3 = (srcp.reshape(NW, NKC1, K1), dstp.reshape(NW, NKC1, K1),
            srcp.reshape(NW, NKC2, K2), dstp.reshape(NW, NKC2, K2))

    x_el = _pad_rows(params["emb"][el])
    x_el = _gat_wide(x_el, idx3, params["element"])
    x_lat = _gat_wide(_pad_rows(lat), idx3, params["latent"])
    x = jnp.concatenate([x_el, x_lat], axis=-1)
    x = _gat_wide(x, idx3, params["pre"])
    for p in params["coords"][:4]:
        x = _gat_wide(x, idx3, p)

    tbl = _tbl_from_x(x.T, params["coords"][4])
    parts = _gat3_parts(tbl, srcp, dstp, params["coords"][4])
    coords_t = _finalize(parts, params["coords"][4]["b"], sig=True)
    coords = coords_t[:3, :N].T

    lengths = _chain3(coords_t, srcp, dstp,
                      [params["post_len"]] + list(params["len_blocks"]))
    lengths = _scatter_mean(lengths[:3, :N].T, batch, N_GRAPHS)
    angles = _chain3(coords_t, srcp, dstp,
                     [params["post_ang"]] + list(params["ang_blocks"]))
    angles = _scatter_mean(angles[:3, :N].T, batch, N_GRAPHS)
    return coords, lengths, angles
